# baseline (speedup 1.0000x reference)
import functools

import jax
import jax.numpy as jnp
from jax import lax
from jax.experimental import pallas as pl
from jax.experimental.pallas import tpu as pltpu
from jax.experimental.pallas import tpu_sc as plsc

B, F, V, D, DD = 16384, 26, 100000, 32, 13
NF = F + DD
NC, NS, L = 2, 16, 16
NW = NC * NS
BPW = B // NW
IPW = BPW * F
BPS = 4
STEP_ROWS = BPS * F
NSTEPS = IPW // STEP_ROWS
G = 8
GROUP_ROWS = G * STEP_ROWS
NG = NSTEPS // G


def _sc_gather_fused(tables_flat, sparse_flat):
    mesh = plsc.VectorSubcoreMesh(core_axis_name="c", subcore_axis_name="s")

    @functools.partial(
        pl.kernel,
        mesh=mesh,
        out_type=jax.ShapeDtypeStruct((B * NF, D), jnp.float32),
        scratch_types=[
            pltpu.VMEM((IPW,), jnp.int32),
            pltpu.VMEM((2, GROUP_ROWS, D), jnp.float32),
            pltpu.SemaphoreType.DMA,
            pltpu.SemaphoreType.DMA,
            pltpu.SemaphoreType.DMA,
            pltpu.SemaphoreType.DMA,
        ],
        compiler_params=pltpu.CompilerParams(use_tc_tiling_on_sc=False),
    )
    def k(tbl_hbm, idx_hbm, out_hbm, idx_v, gbuf, gsem0, gsem1, ssem0, ssem1):
        wid = lax.axis_index("s") * NC + lax.axis_index("c")
        ibase = wid * IPW
        bbase = wid * BPW
        gsems = (gsem0, gsem1)
        ssems = (ssem0, ssem1)

        pltpu.sync_copy(idx_hbm.at[pl.ds(ibase, IPW)], idx_v)

        def conv(i, carry):
            pos = i * L + lax.iota(jnp.int32, L)
            off = lax.rem(pos, F) * V
            idx_v[pl.ds(i * L, L)] = idx_v[pl.ds(i * L, L)] + off
            return carry
        lax.fori_loop(0, IPW // L, conv, None)

        def fire_gathers(g, b):
            for j in range(G):
                pltpu.async_copy(
                    tbl_hbm.at[idx_v.at[pl.ds((g * G + j) * STEP_ROWS,
                                              STEP_ROWS)]],
                    gbuf.at[b, pl.ds(j * STEP_ROWS, STEP_ROWS)],
                    gsems[b])

        def drain_gathers(b):
            pltpu.make_async_copy(
                tbl_hbm.at[pl.ds(0, GROUP_ROWS)], gbuf.at[b],
                gsems[b]).wait()

        def fire_copies(g, b):
            for j in range(G):
                for c in range(BPS):
                    row0 = (bbase + (g * G + j) * BPS + c) * NF
                    pltpu.async_copy(
                        gbuf.at[b, pl.ds(j * STEP_ROWS + c * F, F)],
                        out_hbm.at[pl.ds(row0, F)],
                        ssems[b])

        def drain_copies(b):
            pltpu.make_async_copy(
                tbl_hbm.at[pl.ds(0, GROUP_ROWS)], gbuf.at[b],
                ssems[b]).wait()

        fire_gathers(0, 0)

        def body(gg, carry):
            ga = 2 * gg
            gb = 2 * gg + 1
            drain_gathers(0)

            @pl.when(gg > 0)
            def _():
                drain_copies(1)
            fire_gathers(gb, 1)
            fire_copies(ga, 0)
            drain_gathers(1)

            @pl.when(gg < NG // 2 - 1)
            def _():
                drain_copies(0)
                fire_gathers(ga + 2, 0)
            fire_copies(gb, 1)
            return carry
        lax.fori_loop(0, NG // 2, body, None)
        drain_copies(0)
        drain_copies(1)

    return k(tables_flat, sparse_flat)


def _tc_dense_into(buf, dense_inputs, W3):
    BB = 512

    def mm(_, x_ref, w_ref, o_ref):
        for dd in range(DD):
            o_ref[:, 0, dd, :] = jnp.dot(x_ref[...], w_ref[:, dd, :],
                                         preferred_element_type=jnp.float32)

    return pl.pallas_call(
        mm,
        grid=(B // BB,),
        in_specs=[
            pl.BlockSpec(memory_space=pl.ANY),
            pl.BlockSpec((BB, DD), lambda i: (i, 0)),
            pl.BlockSpec((DD, DD, D), lambda i: (0, 0, 0)),
        ],
        out_specs=pl.BlockSpec((BB, 1, DD, D), lambda i: (i, 2, 0, 0)),
        out_shape=jax.ShapeDtypeStruct((B, 3, DD, D), jnp.float32),
        input_output_aliases={0: 0},
    )(buf, dense_inputs, W3)


def kernel(sparse_inputs, dense_inputs, tables, W):
    tables_flat = tables.reshape(F * V, D)
    sparse_flat = sparse_inputs.reshape(B * F).astype(jnp.int32)
    buf = _sc_gather_fused(tables_flat, sparse_flat).reshape(B, 3, DD, D)
    out = _tc_dense_into(buf, dense_inputs, W.reshape(DD, DD, D))
    return out.reshape(B, NF, D)


# speedup vs baseline: 1.0859x; 1.0859x over previous
import functools

import jax
import jax.numpy as jnp
from jax import lax
from jax.experimental import pallas as pl
from jax.experimental.pallas import tpu as pltpu
from jax.experimental.pallas import tpu_sc as plsc

B, F, V, D, DD = 16384, 26, 100000, 32, 13
NF = F + DD
NC, NS, L = 2, 16, 16
NW = NC * NS
BPW = B // NW
IPW = BPW * F
BPS = 4
STEP_ROWS = BPS * F
G = 8
GROUP_ROWS = G * STEP_ROWS
GB = G * BPS
NG = (BPW // BPS) // G
NB = 8


def _sc_embed(tables_flat, sparse_flat, dense_pad, W):
    mesh = plsc.VectorSubcoreMesh(core_axis_name="c", subcore_axis_name="s")

    @functools.partial(
        pl.kernel,
        mesh=mesh,
        out_type=jax.ShapeDtypeStruct((B, NF, D), jnp.float32),
        scratch_types=[
            pltpu.VMEM((IPW,), jnp.int32),
            pltpu.VMEM((2, GROUP_ROWS, D), jnp.float32),
            pltpu.VMEM((BPW, L), jnp.float32),
            pltpu.VMEM((DD, DD * D), jnp.float32),
            pltpu.VMEM((2, GB, DD, D), jnp.float32),
            pltpu.SemaphoreType.DMA,
            pltpu.SemaphoreType.DMA,
            pltpu.SemaphoreType.DMA,
            pltpu.SemaphoreType.DMA,
            pltpu.SemaphoreType.DMA,
            pltpu.SemaphoreType.DMA,
        ],
        compiler_params=pltpu.CompilerParams(use_tc_tiling_on_sc=False),
    )
    def k(tbl_hbm, idx_hbm, den_hbm, w_hbm, out_hbm,
          idx_v, gbuf, den_v, w_v, dbuf,
          gsem0, gsem1, ssem0, ssem1, dsem0, dsem1):
        wid = lax.axis_index("s") * NC + lax.axis_index("c")
        ibase = wid * IPW
        bbase = wid * BPW
        gsems = (gsem0, gsem1)
        ssems = (ssem0, ssem1)
        dsems = (dsem0, dsem1)

        pltpu.sync_copy(idx_hbm.at[pl.ds(ibase, IPW)], idx_v)
        pltpu.sync_copy(den_hbm.at[pl.ds(bbase, BPW), :], den_v)
        pltpu.sync_copy(w_hbm, w_v)

        def conv(i, carry):
            pos = i * L + lax.iota(jnp.int32, L)
            off = lax.rem(pos, F) * V
            idx_v[pl.ds(i * L, L)] = idx_v[pl.ds(i * L, L)] + off
            return carry
        lax.fori_loop(0, IPW // L, conv, None)

        def fire_gathers(g, b):
            for j in range(G):
                pltpu.async_copy(
                    tbl_hbm.at[idx_v.at[pl.ds((g * G + j) * STEP_ROWS,
                                              STEP_ROWS)]],
                    gbuf.at[b, pl.ds(j * STEP_ROWS, STEP_ROWS)],
                    gsems[b])

        def drain_gathers(b):
            pltpu.make_async_copy(
                tbl_hbm.at[pl.ds(0, GROUP_ROWS)], gbuf.at[b],
                gsems[b]).wait()

        def fire_copies(g, b):
            for j in range(G):
                for c in range(BPS):
                    bb = bbase + (g * G + j) * BPS + c
                    pltpu.async_copy(
                        gbuf.at[b, pl.ds(j * STEP_ROWS + c * F, F)],
                        out_hbm.at[bb, pl.ds(0, F), :],
                        ssems[b])

        def drain_copies(b):
            pltpu.make_async_copy(
                tbl_hbm.at[pl.ds(0, GROUP_ROWS)], gbuf.at[b],
                ssems[b]).wait()

        def dense_compute(g, slot):
            def blk(c0, carry):
                base = g * GB + c0 * NB
                dvecs = [den_v[base + ci, pl.ds(0, L)] for ci in range(NB)]
                d_sc = [[dvecs[ci][kk] for kk in range(DD)]
                        for ci in range(NB)]

                def row(r, carry2):
                    wlo = [w_v[kk, pl.ds(r * D, L)] for kk in range(DD)]
                    whi = [w_v[kk, pl.ds(r * D + L, L)] for kk in range(DD)]
                    for ci in range(NB):
                        acc0 = d_sc[ci][0] * wlo[0]
                        acc1 = d_sc[ci][0] * whi[0]
                        for kk in range(1, DD):
                            acc0 = acc0 + d_sc[ci][kk] * wlo[kk]
                            acc1 = acc1 + d_sc[ci][kk] * whi[kk]
                        c = c0 * NB + ci
                        dbuf[slot, c, r, pl.ds(0, L)] = acc0
                        dbuf[slot, c, r, pl.ds(L, L)] = acc1
                    return carry2
                lax.fori_loop(0, DD, row, None)
                return carry
            lax.fori_loop(0, GB // NB, blk, None)

        def fire_dense_copies(g, slot):
            for c in range(GB):
                bb = bbase + g * GB + c
                pltpu.async_copy(
                    dbuf.at[slot, c],
                    out_hbm.at[bb, pl.ds(F, DD), :],
                    dsems[slot])

        def drain_dense(slot):
            pltpu.make_async_copy(
                out_hbm.at[pl.ds(0, GB), pl.ds(0, DD), :], dbuf.at[slot],
                dsems[slot]).wait()

        fire_gathers(0, 0)

        def body(gg, carry):
            ga = 2 * gg
            gb = 2 * gg + 1
            drain_gathers(0)

            @pl.when(gg > 0)
            def _():
                drain_copies(1)
            fire_gathers(gb, 1)
            fire_copies(ga, 0)

            @pl.when(gg > 0)
            def _():
                drain_dense(0)
            dense_compute(ga, 0)
            fire_dense_copies(ga, 0)

            drain_gathers(1)

            @pl.when(gg < NG // 2 - 1)
            def _():
                drain_copies(0)
                fire_gathers(ga + 2, 0)
            fire_copies(gb, 1)

            @pl.when(gg > 0)
            def _():
                drain_dense(1)
            dense_compute(gb, 1)
            fire_dense_copies(gb, 1)
            return carry
        lax.fori_loop(0, NG // 2, body, None)
        drain_copies(0)
        drain_copies(1)
        drain_dense(0)
        drain_dense(1)

    return k(tables_flat, sparse_flat, dense_pad, W)


def kernel(sparse_inputs, dense_inputs, tables, W):
    tables_flat = tables.reshape(F * V, D)
    sparse_flat = sparse_inputs.reshape(B * F).astype(jnp.int32)
    dense_pad = jnp.pad(dense_inputs, ((0, 0), (0, L - DD)))
    return _sc_embed(tables_flat, sparse_flat, dense_pad, W)
